# baseline (speedup 1.0000x reference)
import functools

import jax
import jax.numpy as jnp
from jax import lax
from jax.experimental import pallas as pl
from jax.experimental.pallas import tpu as pltpu
from jax.experimental.pallas import tpu_sc as plsc

B = 4
S = 8192
D = 128
VOCAB = 100000
N = B * S

NC = 2
NS = 16
NW = NC * NS

TPW = N // NW
CH = 128
NCH = TPW // CH
LANES = 16
VPR = D // LANES


def _body(x_hbm, tok_hbm, pos_hbm, out_hbm, idx_v, tok_v, pos_v, sem):
    c = lax.axis_index("c")
    s = lax.axis_index("s")
    wid = s * NC + c
    base = wid * TPW
    pos_base = lax.rem(base, S)

    for j in range(NCH):
        off = base + j * CH
        pltpu.sync_copy(x_hbm.at[pl.ds(off, CH)], idx_v.at[j])
        gather = pltpu.async_copy(tok_hbm.at[idx_v.at[j]], tok_v, sem)
        pltpu.sync_copy(pos_hbm.at[pl.ds(pos_base + j * CH, CH)], pos_v)
        gather.wait()

        def add_row(r, carry):
            for k in range(VPR):
                tok_v[r, pl.ds(k * LANES, LANES)] += pos_v[r, pl.ds(k * LANES, LANES)]
            return carry

        lax.fori_loop(0, CH, add_row, 0)
        pltpu.sync_copy(tok_v, out_hbm.at[pl.ds(off, CH)])


@jax.jit
def _embed(x_flat, tok_table, pos_table):
    run = pl.kernel(
        _body,
        out_type=jax.ShapeDtypeStruct((N, D), jnp.float32),
        mesh=plsc.VectorSubcoreMesh(
            core_axis_name="c", subcore_axis_name="s",
            num_cores=NC, num_subcores=NS,
        ),
        scratch_types=[
            pltpu.VMEM((NCH, CH), jnp.int32),
            pltpu.VMEM((CH, D), jnp.float32),
            pltpu.VMEM((CH, D), jnp.float32),
            pltpu.SemaphoreType.DMA,
        ],
    )
    return run(x_flat, tok_table, pos_table)


def kernel(x, tok_table, pos_table):
    x_flat = x.reshape(-1).astype(jnp.int32)
    out = _embed(x_flat, tok_table, pos_table)
    return out.reshape(B, S, D)


# speedup vs baseline: 1.5089x; 1.5089x over previous
import jax
import jax.numpy as jnp
from jax import lax
from jax.experimental import pallas as pl
from jax.experimental.pallas import tpu as pltpu
from jax.experimental.pallas import tpu_sc as plsc

B = 4
S = 8192
D = 128
N = B * S

NC = 2
NS = 16
NW = NC * NS

PPW = S // NW
CH = 128
SUBS = PPW // CH
NCH = B * SUBS
NBUF = 4
LANES = 16
VPR = D // LANES


def _body(x_hbm, tok_hbm, pos_hbm, out_hbm, idx_v, pos_v, bufs, gsem, wsem):
    c = lax.axis_index("c")
    s = lax.axis_index("s")
    wid = s * NC + c
    p0 = wid * PPW

    for bt in range(B):
        pltpu.sync_copy(
            x_hbm.at[pl.ds(bt * (S // CH) + wid * SUBS, SUBS)],
            idx_v.at[pl.ds(bt * SUBS, SUBS)],
        )

    def out_row(j):
        bt, sub = j // SUBS, j % SUBS
        return bt * S + p0 + sub * CH

    def start_gather(j):
        return pltpu.async_copy(tok_hbm.at[idx_v.at[j]], bufs.at[j % NBUF], gsem)

    gathers = {j: start_gather(j) for j in range(2)}
    writebacks = {}

    pltpu.sync_copy(pos_hbm.at[pl.ds(p0, PPW)], pos_v)

    for j in range(NCH):
        if j + 2 < NCH:
            if j - 2 >= 0:
                writebacks.pop(j - 2).wait()
            gathers[j + 2] = start_gather(j + 2)
        gathers.pop(j).wait()

        buf = bufs.at[j % NBUF]
        prow = (j % SUBS) * CH

        def add_row(r, carry):
            for k in range(VPR):
                sl = pl.ds(k * LANES, LANES)
                buf[r, sl] += pos_v[prow + r, sl]
            return carry

        lax.fori_loop(0, CH, add_row, 0)
        writebacks[j] = pltpu.async_copy(buf, out_hbm.at[pl.ds(out_row(j), CH)], wsem)

    for j in sorted(writebacks):
        writebacks.pop(j).wait()


@jax.jit
def _embed(x_flat, tok_table, pos_table):
    run = pl.kernel(
        _body,
        out_type=jax.ShapeDtypeStruct((N, D), jnp.float32),
        mesh=plsc.VectorSubcoreMesh(
            core_axis_name="c", subcore_axis_name="s",
            num_cores=NC, num_subcores=NS,
        ),
        scratch_types=[
            pltpu.VMEM((NCH, CH), jnp.int32),
            pltpu.VMEM((PPW, D), jnp.float32),
            pltpu.VMEM((NBUF, CH, D), jnp.float32),
            pltpu.SemaphoreType.DMA,
            pltpu.SemaphoreType.DMA,
        ],
    )
    return run(x_flat.reshape(N // CH, CH), tok_table, pos_table)


def kernel(x, tok_table, pos_table):
    x_flat = x.reshape(-1).astype(jnp.int32)
    out = _embed(x_flat, tok_table, pos_table)
    return out.reshape(B, S, D)
